# baseline (speedup 1.0000x reference)
import functools

import jax
import jax.numpy as jnp
from jax.experimental import pallas as pl
from jax.experimental.pallas import tpu as pltpu

_BM = 128


def _fused_kernel(x_ref, w_ref, b_ref, a0_ref, a1_ref, o_ref, y_ref):
    @pl.when(pl.program_id(0) == 0)
    def _compute_y():
        y_ref[...] = jnp.dot(
            x_ref[...], w_ref[...], preferred_element_type=jnp.float32
        ).astype(jnp.bfloat16)

    bias = b_ref[...].astype(jnp.float32)
    o_ref[0:_BM, :] = bias + jnp.dot(
        a0_ref[...].astype(jnp.bfloat16), y_ref[...],
        preferred_element_type=jnp.float32)
    o_ref[_BM:2 * _BM, :] = bias + jnp.dot(
        a1_ref[...].astype(jnp.bfloat16), y_ref[...],
        preferred_element_type=jnp.float32)


@jax.jit
def kernel(X, A, W, b):
    n, d_in = X.shape
    d_out = W.shape[1]

    b2 = b.reshape(1, d_out)
    grid = (n // (2 * _BM),)

    out = pl.pallas_call(
        _fused_kernel,
        grid=grid,
        in_specs=[
            pl.BlockSpec((n, d_in), lambda i: (0, 0)),
            pl.BlockSpec((d_in, d_out), lambda i: (0, 0)),
            pl.BlockSpec((1, d_out), lambda i: (0, 0)),
            pl.BlockSpec((_BM, n), lambda i: (2 * i, 0)),
            pl.BlockSpec((_BM, n), lambda i: (2 * i + 1, 0)),
        ],
        out_specs=pl.BlockSpec((2 * _BM, d_out), lambda i: (i, 0)),
        out_shape=jax.ShapeDtypeStruct((n, d_out), jnp.float32),
        scratch_shapes=[pltpu.VMEM((n, d_out), jnp.bfloat16)],
        compiler_params=pltpu.CompilerParams(
            dimension_semantics=("arbitrary",),
        ),
    )(X, W, b2, A, A)
    return out


# speedup vs baseline: 1.0112x; 1.0112x over previous
import functools

import jax
import jax.numpy as jnp
from jax.experimental import pallas as pl
from jax.experimental.pallas import tpu as pltpu

_BM = 64
_NBUF = 8
_UNROLL = 4


def _fused_kernel(x_ref, w_ref, b_ref, a_hbm, o_ref, y_ref, abuf, sem):
    n = x_ref.shape[0]
    nblk = n // _BM

    y_ref[...] = jnp.dot(
        x_ref[...], w_ref[...], preferred_element_type=jnp.float32
    ).astype(jnp.bfloat16)

    def _copy(blk):
        slot = jax.lax.rem(blk, _NBUF)
        return pltpu.make_async_copy(
            a_hbm.at[pl.ds(blk * _BM, _BM), :],
            abuf.at[slot],
            sem.at[slot],
        )

    for blk in range(_NBUF):
        _copy(blk).start()

    bias = b_ref[...].astype(jnp.float32)

    def _body(t, carry):
        for j in range(_UNROLL):
            blk = t * _UNROLL + j
            _copy(blk).wait()
            slot = jax.lax.rem(blk, _NBUF)
            acc = jnp.dot(abuf[slot].astype(jnp.bfloat16), y_ref[...],
                          preferred_element_type=jnp.float32)
            o_ref[pl.ds(blk * _BM, _BM), :] = acc + bias

            @pl.when(blk + _NBUF < nblk)
            def _prefetch():
                _copy(blk + _NBUF).start()

        return carry

    jax.lax.fori_loop(0, nblk // _UNROLL, _body, 0)


@jax.jit
def kernel(X, A, W, b):
    n, d_in = X.shape
    d_out = W.shape[1]

    b2 = b.reshape(1, d_out)
    out = pl.pallas_call(
        _fused_kernel,
        grid=(1,),
        in_specs=[
            pl.BlockSpec((n, d_in), lambda i: (0, 0)),
            pl.BlockSpec((d_in, d_out), lambda i: (0, 0)),
            pl.BlockSpec((1, d_out), lambda i: (0, 0)),
            pl.BlockSpec(memory_space=pltpu.MemorySpace.HBM),
        ],
        out_specs=pl.BlockSpec((n, d_out), lambda i: (0, 0)),
        out_shape=jax.ShapeDtypeStruct((n, d_out), jnp.float32),
        scratch_shapes=[
            pltpu.VMEM((n, d_out), jnp.bfloat16),
            pltpu.VMEM((_NBUF, _BM, n), jnp.float32),
            pltpu.SemaphoreType.DMA((_NBUF,)),
        ],
        compiler_params=pltpu.CompilerParams(
            dimension_semantics=("arbitrary",),
        ),
    )(X, W, b2, A)
    return out


# speedup vs baseline: 1.0200x; 1.0087x over previous
import functools

import jax
import jax.numpy as jnp
from jax.experimental import pallas as pl
from jax.experimental.pallas import tpu as pltpu

_BM = 128
_NSPLIT = 4


def _fused_kernel(x_ref, w_ref, b_ref, *rest):
    a_refs = rest[:_NSPLIT]
    o_ref = rest[_NSPLIT]
    y_ref = rest[_NSPLIT + 1]

    @pl.when(pl.program_id(0) == 0)
    def _compute_y():
        y_ref[...] = jnp.dot(
            x_ref[...], w_ref[...], preferred_element_type=jnp.float32
        ).astype(jnp.bfloat16)

    h = a_refs[0].shape[1]
    acc = b_ref[...].astype(jnp.float32)
    for j, a_ref in enumerate(a_refs):
        acc += jnp.dot(a_ref[...].astype(jnp.bfloat16),
                       y_ref[j * h:(j + 1) * h, :],
                       preferred_element_type=jnp.float32)
    o_ref[...] = acc


@jax.jit
def kernel(X, A, W, b):
    n, d_in = X.shape
    d_out = W.shape[1]

    b2 = b.reshape(1, d_out)
    h = n // _NSPLIT
    grid = (n // _BM,)

    def _a_spec(j):
        return pl.BlockSpec((_BM, h), lambda i, j=j: (i, j))

    out = pl.pallas_call(
        _fused_kernel,
        grid=grid,
        in_specs=[
            pl.BlockSpec((n, d_in), lambda i: (0, 0)),
            pl.BlockSpec((d_in, d_out), lambda i: (0, 0)),
            pl.BlockSpec((1, d_out), lambda i: (0, 0)),
        ] + [_a_spec(j) for j in range(_NSPLIT)],
        out_specs=pl.BlockSpec((_BM, d_out), lambda i: (i, 0)),
        out_shape=jax.ShapeDtypeStruct((n, d_out), jnp.float32),
        scratch_shapes=[pltpu.VMEM((n, d_out), jnp.bfloat16)],
        compiler_params=pltpu.CompilerParams(
            dimension_semantics=("arbitrary",),
        ),
    )(X, W, b2, *([A] * _NSPLIT))
    return out


# speedup vs baseline: 1.0238x; 1.0037x over previous
import functools

import jax
import jax.numpy as jnp
from jax.experimental import pallas as pl
from jax.experimental.pallas import tpu as pltpu

_BM = 128


def _fused_kernel(x_ref, w_ref, b_ref, a_auto_ref, a_hbm, o_ref,
                  y_ref, abuf, sem):
    n = x_ref.shape[0]
    h = n // 2
    nblk = n // _BM
    i = pl.program_id(0)

    @pl.when(i == 0)
    def _first():
        y_ref[...] = jnp.dot(
            x_ref[...], w_ref[...], preferred_element_type=jnp.float32
        ).astype(jnp.bfloat16)

    def _copy(blk):
        slot = jax.lax.rem(blk, 2)
        return pltpu.make_async_copy(
            a_hbm.at[pl.ds(blk * _BM, _BM), pl.ds(h, h)],
            abuf.at[slot],
            sem.at[slot],
        )

    @pl.when(i == 0)
    def _prologue():
        _copy(0).start()

    @pl.when(i + 1 < nblk)
    def _prefetch():
        _copy(i + 1).start()

    _copy(i).wait()
    slot = jax.lax.rem(i, 2)
    acc = b_ref[...].astype(jnp.float32)
    acc += jnp.dot(a_auto_ref[...].astype(jnp.bfloat16), y_ref[0:h, :],
                   preferred_element_type=jnp.float32)
    acc += jnp.dot(abuf[slot].astype(jnp.bfloat16), y_ref[h:n, :],
                   preferred_element_type=jnp.float32)
    o_ref[...] = acc


@jax.jit
def kernel(X, A, W, b):
    n, d_in = X.shape
    d_out = W.shape[1]

    b2 = b.reshape(1, d_out)
    h = n // 2
    out = pl.pallas_call(
        _fused_kernel,
        grid=(n // _BM,),
        in_specs=[
            pl.BlockSpec((n, d_in), lambda i: (0, 0)),
            pl.BlockSpec((d_in, d_out), lambda i: (0, 0)),
            pl.BlockSpec((1, d_out), lambda i: (0, 0)),
            pl.BlockSpec((_BM, h), lambda i: (i, 0)),
            pl.BlockSpec(memory_space=pltpu.MemorySpace.HBM),
        ],
        out_specs=pl.BlockSpec((_BM, d_out), lambda i: (i, 0)),
        out_shape=jax.ShapeDtypeStruct((n, d_out), jnp.float32),
        scratch_shapes=[
            pltpu.VMEM((n, d_out), jnp.bfloat16),
            pltpu.VMEM((2, _BM, h), jnp.float32),
            pltpu.SemaphoreType.DMA((2,)),
        ],
        compiler_params=pltpu.CompilerParams(
            dimension_semantics=("arbitrary",),
        ),
    )(X, W, b2, A, A)
    return out
